# baseline (speedup 1.0000x reference)
import functools

import jax
import jax.numpy as jnp
from jax import lax
from jax.experimental import pallas as pl
from jax.experimental.pallas import tpu as pltpu
from jax.experimental.pallas import tpu_sc as plsc

_NC = 2
_NS = 16
_NW = _NC * _NS
_L = 16


@functools.partial(jax.jit, static_argnums=(2, 3, 4))
def _embed_gather(table, idx, b, s, d):
    n = b * s
    b_per_w = n // _NW
    chunk = 64
    n_ch = b_per_w // chunk
    nb = 2

    @functools.partial(
        pl.kernel,
        mesh=plsc.VectorSubcoreMesh(core_axis_name="c", subcore_axis_name="s"),
        out_type=jax.ShapeDtypeStruct((n, d), jnp.float32),
        scratch_types=(
            [pltpu.VMEM((b_per_w,), jnp.int32)]
            + [pltpu.VMEM((n_ch, chunk), jnp.int32)]
            + [pltpu.VMEM((chunk, d), jnp.float32) for _ in range(nb)]
            + [pltpu.SemaphoreType.DMA for _ in range(2 * nb)]
        ),
    )
    def k(table_hbm, idx_hbm, out_hbm, idx_v, widx_v, *rest):
        bufs = rest[:nb]
        gsems = rest[nb:2 * nb]
        wsems = rest[2 * nb:]
        wid = lax.axis_index("s") * _NC + lax.axis_index("c")
        base = wid * b_per_w
        pltpu.sync_copy(idx_hbm.at[pl.ds(base, b_per_w)], idx_v)

        lane = lax.iota(jnp.int32, _L)
        for c in range(n_ch):
            for j in range(chunk // _L):
                widx_v[c, pl.ds(j * _L, _L)] = lane + (base + c * chunk + j * _L)

        def write(c, buf, sem):
            if c % 2 == 0:
                return pltpu.async_copy(
                    buf, out_hbm.at[pl.ds(base + c * chunk, chunk)], sem)
            return pltpu.async_copy(buf, out_hbm.at[widx_v.at[c]], sem)

        gcop = [None] * n_ch
        wcop = [None] * n_ch
        for c in range(n_ch):
            bi = c % nb
            if c >= nb:
                wcop[c - nb].wait()
            gcop[c] = pltpu.async_copy(
                table_hbm.at[idx_v.at[pl.ds(c * chunk, chunk)]],
                bufs[bi],
                gsems[bi],
            )
            if c >= 1:
                gcop[c - 1].wait()
                wcop[c - 1] = write(c - 1, bufs[(c - 1) % nb], wsems[(c - 1) % nb])
        gcop[n_ch - 1].wait()
        wcop[n_ch - 1] = write(n_ch - 1, bufs[(n_ch - 1) % nb],
                               wsems[(n_ch - 1) % nb])
        for c in range(max(0, n_ch - nb), n_ch):
            wcop[c].wait()

    return k(table, idx)


def kernel(inputs, embed_table):
    b, s = inputs.shape
    v, d = embed_table.shape
    out = _embed_gather(embed_table, inputs.reshape(b * s).astype(jnp.int32),
                        b, s, d)
    return out.reshape(b, s, d)


# speedup vs baseline: 1.0126x; 1.0126x over previous
import functools

import jax
import jax.numpy as jnp
from jax import lax
from jax.experimental import pallas as pl
from jax.experimental.pallas import tpu as pltpu
from jax.experimental.pallas import tpu_sc as plsc

_NC = 2
_NS = 16
_NW = _NC * _NS


@functools.partial(jax.jit, static_argnums=(2, 3))
def _gather_rows(table, idx, n, d):
    b_per_w = n // _NW
    chunk = 64
    n_ch = b_per_w // chunk

    nb = 2

    @functools.partial(
        pl.kernel,
        mesh=plsc.VectorSubcoreMesh(core_axis_name="c", subcore_axis_name="s"),
        out_type=jax.ShapeDtypeStruct((n, d), jnp.float32),
        scratch_types=(
            [pltpu.VMEM((b_per_w,), jnp.int32)]
            + [pltpu.VMEM((chunk, d), jnp.float32) for _ in range(nb)]
            + [pltpu.SemaphoreType.DMA for _ in range(2 * nb)]
        ),
    )
    def k(table_hbm, idx_hbm, out_hbm, idx_v, *rest):
        bufs = rest[:nb]
        gsems = rest[nb:2 * nb]
        wsems = rest[2 * nb:]
        wid = lax.axis_index("s") * _NC + lax.axis_index("c")
        base = wid * b_per_w
        pltpu.sync_copy(idx_hbm.at[pl.ds(base, b_per_w)], idx_v)

        gcop = [None] * n_ch
        wcop = [None] * n_ch
        for c in range(n_ch):
            b = c % nb
            if c >= nb:
                wcop[c - nb].wait()
            gcop[c] = pltpu.async_copy(
                table_hbm.at[idx_v.at[pl.ds(c * chunk, chunk)]],
                bufs[b],
                gsems[b],
            )
            if c >= 1:
                gcop[c - 1].wait()
                wcop[c - 1] = pltpu.async_copy(
                    bufs[(c - 1) % nb],
                    out_hbm.at[pl.ds(base + (c - 1) * chunk, chunk)],
                    wsems[(c - 1) % nb],
                )
        gcop[n_ch - 1].wait()
        wcop[n_ch - 1] = pltpu.async_copy(
            bufs[(n_ch - 1) % nb],
            out_hbm.at[pl.ds(base + (n_ch - 1) * chunk, chunk)],
            wsems[(n_ch - 1) % nb],
        )
        for c in range(max(0, n_ch - nb), n_ch):
            wcop[c].wait()

    return k(table, idx)


def kernel(inputs, embed_table):
    b, s = inputs.shape
    v, d = embed_table.shape
    n = b * s
    idx = inputs.reshape(n).astype(jnp.int32)
    out = _gather_rows(embed_table, idx, n, d)
    return out.reshape(b, s, d)


# speedup vs baseline: 1.0161x; 1.0034x over previous
import functools

import jax
import jax.numpy as jnp
from jax import lax
from jax.experimental import pallas as pl
from jax.experimental.pallas import tpu as pltpu
from jax.experimental.pallas import tpu_sc as plsc

_NC = 2
_NS = 16
_NW = _NC * _NS


@functools.partial(jax.jit, static_argnums=(2, 3))
def _gather_rows(table, idx, n, d):
    b_per_w = n // _NW
    chunk = 64
    n_ch = b_per_w // chunk

    nb = 2

    @functools.partial(
        pl.kernel,
        mesh=plsc.VectorSubcoreMesh(core_axis_name="c", subcore_axis_name="s"),
        out_type=jax.ShapeDtypeStruct((n, d), jnp.float32),
        compiler_params=pltpu.CompilerParams(use_tc_tiling_on_sc=True),
        scratch_types=(
            [pltpu.VMEM((b_per_w,), jnp.int32)]
            + [pltpu.VMEM((chunk, d), jnp.float32) for _ in range(nb)]
            + [pltpu.SemaphoreType.DMA for _ in range(2 * nb)]
        ),
    )
    def k(table_hbm, idx_hbm, out_hbm, idx_v, *rest):
        bufs = rest[:nb]
        gsems = rest[nb:2 * nb]
        wsems = rest[2 * nb:]
        wid = lax.axis_index("s") * _NC + lax.axis_index("c")
        base = wid * b_per_w
        pltpu.sync_copy(idx_hbm.at[pl.ds(base, b_per_w)], idx_v)

        gcop = [None] * n_ch
        wcop = [None] * n_ch
        for c in range(n_ch):
            b = c % nb
            if c >= nb:
                wcop[c - nb].wait()
            gcop[c] = pltpu.async_copy(
                table_hbm.at[idx_v.at[pl.ds(c * chunk, chunk)]],
                bufs[b],
                gsems[b],
            )
            if c >= 1:
                gcop[c - 1].wait()
                wcop[c - 1] = pltpu.async_copy(
                    bufs[(c - 1) % nb],
                    out_hbm.at[pl.ds(base + (c - 1) * chunk, chunk)],
                    wsems[(c - 1) % nb],
                )
        gcop[n_ch - 1].wait()
        wcop[n_ch - 1] = pltpu.async_copy(
            bufs[(n_ch - 1) % nb],
            out_hbm.at[pl.ds(base + (n_ch - 1) * chunk, chunk)],
            wsems[(n_ch - 1) % nb],
        )
        for c in range(max(0, n_ch - nb), n_ch):
            wcop[c].wait()

    return k(table, idx)


def kernel(inputs, embed_table):
    b, s = inputs.shape
    v, d = embed_table.shape
    n = b * s
    idx = inputs.reshape(n).astype(jnp.int32)
    out = _gather_rows(embed_table, idx, n, d)
    return out.reshape(b, s, d)
